# Initial kernel scaffold; baseline (speedup 1.0000x reference)
import jax
import jax.numpy as jnp
from jax.experimental import pallas as pl


def kernel(embedded_features, actor_Ws, actor_bs, critic_Ws, critic_bs):
    raise NotImplementedError("write your pallas kernel here")


# speedup vs baseline: 232.1376x; 232.1376x over previous
import functools

import jax
import jax.numpy as jnp
from jax.experimental import pallas as pl

_NEG = -1e30


def _mlpx_kernel(nb, g_ref, n_ref, nT_ref, m_ref, wg_ref, w1_ref, w2t_ref,
                 b0_ref, wl_ref, bl_ref, cw0_ref, cb0_ref, cw1_ref, cb1_ref,
                 out_ref, val_ref):
    f32 = jnp.float32
    counts = jnp.sum(jnp.sum(m_ref[...], axis=2), axis=1)
    maxc = jnp.max(counts)

    wl_row = wl_ref[...]
    bl = bl_ref[0, 0]
    c0 = jnp.sum(jnp.tanh(b0_ref[...]) * wl_row) + bl

    gW = jnp.dot(g_ref[...], wg_ref[...],
                 preferred_element_type=f32) + b0_ref[...]

    for b in range(nb):
        a_b = jnp.dot(n_ref[b], w1_ref[...],
                      preferred_element_type=f32) + gW[b:b + 1]
        bT_b = jnp.dot(w2t_ref[...], nT_ref[b],
                       preferred_element_type=f32)
        z = jnp.tanh(a_b[:, :, None] + bT_b[None, :, :])
        s_b = jnp.sum(z * wl_row[:, :, None], axis=1) + bl
        m_b = m_ref[b]
        mv = jnp.max(jnp.where(m_b > 0, s_b, _NEG))
        mx = jnp.maximum(mv, jnp.where(counts[b] < maxc, c0, _NEG))
        e = jnp.exp(s_b - mx)
        denom = (jnp.sum(jnp.where(m_b > 0, e, 0.0))
                 + (maxc - counts[b]) * jnp.exp(c0 - mx))
        out_ref[b] = jnp.where(m_b > 0, e / denom, 0.0)

    h = jnp.tanh(jnp.dot(g_ref[...], cw0_ref[...],
                         preferred_element_type=f32) + cb0_ref[...])
    val_ref[...] = jnp.dot(h, cw1_ref[...],
                           preferred_element_type=f32) + cb1_ref[...]


def kernel(embedded_features, actor_Ws, actor_bs, critic_Ws, critic_bs):
    b, np1, f = embedded_features.shape
    h = critic_Ws[0].shape[0]
    n = np1 - 1

    g = embedded_features[:, 0, :h]
    n_emb = embedded_features[:, 1:, :h]
    n_embT = jnp.swapaxes(n_emb, 1, 2)
    mask = embedded_features[:, 1:, h:]

    w0, wl = actor_Ws
    b0, bl = actor_bs
    wg, w1, w2 = w0[:h], w0[h:2 * h], w0[2 * h:]
    k = w0.shape[1]

    out, val = pl.pallas_call(
        functools.partial(_mlpx_kernel, b),
        out_shape=[
            jax.ShapeDtypeStruct((b, n, n), jnp.float32),
            jax.ShapeDtypeStruct((b, 1), jnp.float32),
        ],
    )(g, n_emb, n_embT, mask, wg, w1, w2.T,
      b0.reshape(1, k), wl.reshape(1, k), bl.reshape(1, 1),
      critic_Ws[0], critic_bs[0].reshape(1, k), critic_Ws[1],
      critic_bs[1].reshape(1, 1))

    return out.reshape(b, n * n), val.reshape(b, 1, 1)


# speedup vs baseline: 265.3583x; 1.1431x over previous
import functools

import jax
import jax.numpy as jnp
from jax.experimental import pallas as pl

_NEG = -1e30


def _mlpx_kernel(nb, h, ef_ref, wg_ref, w1_ref, w2t_ref,
                 b0_ref, wl_ref, bl_ref, cw0_ref, cb0_ref, cw1_ref, cb1_ref,
                 out_ref, val_ref):
    f32 = jnp.float32
    k = w1_ref.shape[1]

    counts = [jnp.sum(ef_ref[b, 1:, h:]) for b in range(nb)]
    maxc = functools.reduce(jnp.maximum, counts)

    wl_row = wl_ref[...]
    bl = bl_ref[0, 0]
    c0 = jnp.sum(jnp.tanh(b0_ref[...]) * wl_row) + bl

    g_all = ef_ref[:, 0, :h]
    gW = jnp.dot(g_all, wg_ref[...],
                 preferred_element_type=f32) + b0_ref[...]

    for b in range(nb):
        ne_b = ef_ref[b, 1:, :h]
        a_b = jnp.dot(ne_b, w1_ref[...],
                      preferred_element_type=f32) + gW[b:b + 1]
        bT_b = jax.lax.dot_general(w2t_ref[...], ne_b,
                                   (((1,), (1,)), ((), ())),
                                   preferred_element_type=f32)
        s_b = jnp.full(out_ref.shape[1:], bl, f32)
        for kk in range(k):
            z = jnp.tanh(a_b[:, kk:kk + 1] + bT_b[kk:kk + 1, :])
            s_b = s_b + wl_ref[0, kk] * z
        m_b = ef_ref[b, 1:, h:]
        mv = jnp.max(jnp.where(m_b > 0, s_b, _NEG))
        mx = jnp.maximum(mv, jnp.where(counts[b] < maxc, c0, _NEG))
        e = jnp.exp(s_b - mx)
        inv = 1.0 / (jnp.sum(jnp.where(m_b > 0, e, 0.0))
                     + (maxc - counts[b]) * jnp.exp(c0 - mx))
        out_ref[b] = jnp.where(m_b > 0, e * inv, 0.0)

    hh = jnp.tanh(jnp.dot(g_all, cw0_ref[...],
                          preferred_element_type=f32) + cb0_ref[...])
    val_ref[...] = jnp.dot(hh, cw1_ref[...],
                           preferred_element_type=f32) + cb1_ref[...]


def kernel(embedded_features, actor_Ws, actor_bs, critic_Ws, critic_bs):
    b, np1, f = embedded_features.shape
    h = critic_Ws[0].shape[0]
    n = np1 - 1

    w0, wl = actor_Ws
    b0, bl = actor_bs
    wg, w1, w2 = w0[:h], w0[h:2 * h], w0[2 * h:]
    k = w0.shape[1]

    out, val = pl.pallas_call(
        functools.partial(_mlpx_kernel, b, h),
        out_shape=[
            jax.ShapeDtypeStruct((b, n, n), jnp.float32),
            jax.ShapeDtypeStruct((b, 1), jnp.float32),
        ],
    )(embedded_features, wg, w1, w2.T,
      b0.reshape(1, k), wl.reshape(1, k), bl.reshape(1, 1),
      critic_Ws[0], critic_bs[0].reshape(1, k), critic_Ws[1],
      critic_bs[1].reshape(1, 1))

    return out.reshape(b, n * n), val.reshape(b, 1, 1)


# speedup vs baseline: 270.4928x; 1.0193x over previous
import functools

import jax
import jax.numpy as jnp
from jax.experimental import pallas as pl

_NEG = -1e30


def _mlpx_kernel(nb, h, ef_ref, wg_ref, w1_ref, w2t_ref,
                 b0_ref, wl_ref, bl_ref, cw0_ref, cb0_ref, cw1_ref, cb1_ref,
                 out_ref, val_ref):
    f32 = jnp.float32
    k = w1_ref.shape[1]

    counts = [jnp.sum(ef_ref[b, 1:, h:]) for b in range(nb)]
    maxc = functools.reduce(jnp.maximum, counts)

    wl_row = wl_ref[...]
    bl = bl_ref[0, 0]
    c0 = jnp.sum(jnp.tanh(b0_ref[...]) * wl_row) + bl

    g_all = ef_ref[:, 0, :h]
    gW = jnp.dot(g_all, wg_ref[...],
                 preferred_element_type=f32) + b0_ref[...]

    for b in range(nb):
        ne_b = ef_ref[b, 1:, :h]
        a_b = jnp.dot(ne_b, w1_ref[...],
                      preferred_element_type=f32) + gW[b:b + 1]
        bT_b = jax.lax.dot_general(w2t_ref[...], ne_b,
                                   (((1,), (1,)), ((), ())),
                                   preferred_element_type=f32)
        s_b = jnp.full(out_ref.shape[1:], bl, f32)
        for kk in range(k):
            z = jnp.tanh(a_b[:, kk:kk + 1] + bT_b[kk:kk + 1, :])
            s_b = s_b + wl_ref[0, kk] * z
        m_b = ef_ref[b, 1:, h:]
        mv = jnp.max(jnp.where(m_b > 0, s_b, _NEG))
        mx = jnp.maximum(mv, jnp.where(counts[b] < maxc, c0, _NEG))
        e_m = jnp.where(m_b > 0, jnp.exp(s_b - mx), 0.0)
        denom = jnp.sum(e_m) + (maxc - counts[b]) * jnp.exp(c0 - mx)
        inv = jnp.where(denom > 0, 1.0 / denom, 0.0)
        out_ref[b] = e_m * inv

    hh = jnp.tanh(jnp.dot(g_all, cw0_ref[...],
                          preferred_element_type=f32) + cb0_ref[...])
    val_ref[...] = jnp.dot(hh, cw1_ref[...],
                           preferred_element_type=f32) + cb1_ref[...]


def kernel(embedded_features, actor_Ws, actor_bs, critic_Ws, critic_bs):
    b, np1, f = embedded_features.shape
    h = critic_Ws[0].shape[0]
    n = np1 - 1

    w0, wl = actor_Ws
    b0, bl = actor_bs
    wg, w1, w2 = w0[:h], w0[h:2 * h], w0[2 * h:]
    k = w0.shape[1]

    out, val = pl.pallas_call(
        functools.partial(_mlpx_kernel, b, h),
        out_shape=[
            jax.ShapeDtypeStruct((b, n, n), jnp.float32),
            jax.ShapeDtypeStruct((b, 1), jnp.float32),
        ],
    )(embedded_features, wg, w1, w2.T,
      b0.reshape(1, k), wl.reshape(1, k), bl.reshape(1, 1),
      critic_Ws[0], critic_bs[0].reshape(1, k), critic_Ws[1],
      critic_bs[1].reshape(1, 1))

    return out.reshape(b, n * n), val.reshape(b, 1, 1)
